# Initial kernel scaffold; baseline (speedup 1.0000x reference)
import jax
import jax.numpy as jnp
from jax.experimental import pallas as pl


def kernel(C, Fn, A, SP1, SP0, W1_w, W1_b, W2_w, W2_b, W5_w, W5_b, W6_w, W6_b, c1_sw, c1_nw, c1_b, c2_sw, c2_nw, c2_b, W9_w, W9_b, W10_w, W10_b, edge_index):
    raise NotImplementedError("write your pallas kernel here")


# speedup vs baseline: 1.9835x; 1.9835x over previous
import functools

import jax
import jax.numpy as jnp
from jax import lax
from jax.experimental import pallas as pl
from jax.experimental.pallas import tpu as pltpu
from jax.experimental.pallas import tpu_sc as plsc

_N = 10000
_E = 160000
_D = 256
_H = 128
_NP = 10240
_RPS = 640
_NSUB = 16
_EPS = _E // _NSUB
_CK = 80
_NCH = _EPS // _CK
_EPW = _E // 32
_CKD = 40
_NCHD = _EPW // _CKD

_f32 = jnp.float32


def _enc2_body(x0, x1, wc, b1, w2t, b2, lo, hi):
    h = x0[...] * wc[0:1, :] + x1[...] * wc[1:2, :] + b1[...]
    h = jnp.maximum(h, 0.0)
    o = jnp.dot(h, w2t[...], preferred_element_type=_f32) + b2[...]
    lo[...] = o[:, :_H]
    hi[...] = o[:, _H:]


def _enc3_body(x0, x1, x2, wc, b1, w2t, b2, lo, hi):
    h = (x0[...] * wc[0:1, :] + x1[...] * wc[1:2, :] + x2[...] * wc[2:3, :]
         + b1[...])
    h = jnp.maximum(h, 0.0)
    o = jnp.dot(h, w2t[...], preferred_element_type=_f32) + b2[...]
    lo[...] = o[:, :_H]
    hi[...] = o[:, _H:]


def _node1_body(nlo, nhi, rlo, rhi, dg, swt, nwt, b1, nw2t, sw2t, b2,
                m2lo, m2hi, s2):
    n = jnp.concatenate([nlo[...], nhi[...]], axis=1)
    raw = jnp.concatenate([rlo[...], rhi[...]], axis=1)
    deg = jnp.maximum(dg[:, :1], 1.0)
    neigh = jnp.dot(raw / deg, nwt[...], preferred_element_type=_f32)
    h1 = jnp.maximum(
        jnp.dot(n, swt[...], preferred_element_type=_f32) + neigh + b1[...],
        0.0)
    hcat = jnp.concatenate([h1, n], axis=1)
    m2 = jnp.dot(hcat, nw2t[...], preferred_element_type=_f32)
    m2lo[...] = m2[:, :_H]
    m2hi[...] = m2[:, _H:]
    s2[...] = jnp.dot(hcat, sw2t[...], preferred_element_type=_f32) + b2[...]


def _node2_body(s2, rlo, rhi, dg, nlo, nhi, nw2t, sw2t, b2,
                m3lo, m3hi, s3):
    raw = jnp.concatenate([rlo[...], rhi[...]], axis=1)
    deg = jnp.maximum(dg[:, :1], 1.0)
    h2 = jnp.maximum(s2[...] + raw / deg, 0.0)
    n = jnp.concatenate([nlo[...], nhi[...]], axis=1)
    hcat = jnp.concatenate([h2, n], axis=1)
    m3 = jnp.dot(hcat, nw2t[...], preferred_element_type=_f32)
    m3lo[...] = m3[:, :_H]
    m3hi[...] = m3[:, _H:]
    s3[...] = jnp.dot(hcat, sw2t[...], preferred_element_type=_f32) + b2[...]


def _node3_body(s3, rlo, rhi, dg, w9at, w9bt, b9,
                palo, pahi, pblo, pbhi):
    raw = jnp.concatenate([rlo[...], rhi[...]], axis=1)
    deg = jnp.maximum(dg[:, :1], 1.0)
    h3 = s3[...] + raw / deg
    pa = jnp.dot(h3, w9at[...], preferred_element_type=_f32) + b9[...]
    pb = jnp.dot(h3, w9bt[...], preferred_element_type=_f32)
    palo[...] = pa[:, :_H]
    pahi[...] = pa[:, _H:]
    pblo[...] = pb[:, :_H]
    pbhi[...] = pb[:, _H:]


def _final_body(p0, p1, b10, o):
    o[...] = jnp.sum(p0[...] + p1[...], axis=1, keepdims=True) + b10[0, 0]


def _full(shape):
    return pl.BlockSpec(shape, lambda i: tuple(0 for _ in shape))


def _rows(bs, w):
    return pl.BlockSpec((bs, w), lambda i: (i, 0))


def _deg_body(dst, ones_h, out0, acc, zvm, ones_v, idx_d):
    cid = lax.axis_index("c")
    sid = lax.axis_index("s")

    def zrow(i, _):
        for j in range(8):
            zvm[i, pl.ds(j * 16, 16)] = jnp.zeros((16,), _f32)
        return 0
    lax.fori_loop(0, 128, zrow, 0)
    pltpu.sync_copy(ones_h, ones_v)

    def zacc(k, _):
        pltpu.sync_copy(zvm, acc.at[pl.ds(sid * _RPS + k * 128, 128)])
        return 0
    lax.fori_loop(0, 5, zacc, 0)
    plsc.subcore_barrier()

    def chunk(ch, _):
        base = sid * _EPS + ch * _CK
        pltpu.sync_copy(dst.at[pl.ds(base, _CK)], idx_d)
        pltpu.sync_copy(ones_v, acc.at[idx_d], add=True)
        return 0
    lax.fori_loop(0, _NCH, chunk, 0)
    plsc.subcore_barrier()

    @pl.when(cid == 0)
    def _():
        pltpu.sync_copy(acc.at[pl.ds(sid * _RPS, _RPS)],
                        out0.at[pl.ds(sid * _RPS, _RPS)])


def _seg_body(flo, fhi, elo, ehi, src, dst, out_lo, out_hi,
              acc, rows_v, e_v, idx_s, idx_d, zvm, sem_g, sem_e):
    cid = lax.axis_index("c")
    sid = lax.axis_index("s")

    def zrow(i, _):
        for j in range(8):
            zvm[i, pl.ds(j * 16, 16)] = jnp.zeros((16,), _f32)
        return 0
    lax.fori_loop(0, 128, zrow, 0)

    def zacc(k, _):
        pltpu.sync_copy(zvm, acc.at[pl.ds(sid * _RPS + k * 128, 128)])
        return 0
    lax.fori_loop(0, 5, zacc, 0)
    plsc.subcore_barrier()

    def run(feat_ref, e_ref):
        def chunk(ch, _):
            base = sid * _EPS + ch * _CK
            pltpu.sync_copy(src.at[pl.ds(base, _CK)], idx_s)
            pltpu.sync_copy(dst.at[pl.ds(base, _CK)], idx_d)
            cg = pltpu.async_copy(feat_ref.at[idx_s], rows_v, sem_g)
            ce = pltpu.async_copy(e_ref.at[pl.ds(base, _CK)], e_v, sem_e)
            cg.wait()
            ce.wait()

            def mul(i, _):
                for j in range(8):
                    s = pl.ds(j * 16, 16)
                    rows_v[i, s] = rows_v[i, s] * e_v[i, s]
                return 0
            lax.fori_loop(0, _CK, mul, 0)
            pltpu.sync_copy(rows_v, acc.at[idx_d], add=True)
            return 0
        lax.fori_loop(0, _NCH, chunk, 0)

    @pl.when(cid == 0)
    def _():
        run(flo, elo)

    @pl.when(cid == 1)
    def _():
        run(fhi, ehi)

    plsc.subcore_barrier()

    @pl.when(cid == 0)
    def _():
        pltpu.sync_copy(acc.at[pl.ds(sid * _RPS, _RPS)],
                        out_lo.at[pl.ds(sid * _RPS, _RPS)])

    @pl.when(cid == 1)
    def _():
        pltpu.sync_copy(acc.at[pl.ds(sid * _RPS, _RPS)],
                        out_hi.at[pl.ds(sid * _RPS, _RPS)])


def _edge_body(palo, pahi, pblo, pbhi, w10lo, w10hi, src, dst, out0, out1,
               a_v, b_v, w_v, o_v, idx_s, idx_d, sem_a, sem_b):
    cid = lax.axis_index("c")
    sid = lax.axis_index("s")

    def run(pa_ref, pb_ref, w_ref, out_ref):
        pltpu.sync_copy(w_ref, w_v)

        def chunk(ch, _):
            base = sid * _EPS + ch * _CK
            pltpu.sync_copy(src.at[pl.ds(base, _CK)], idx_s)
            pltpu.sync_copy(dst.at[pl.ds(base, _CK)], idx_d)
            ca = pltpu.async_copy(pa_ref.at[idx_s], a_v, sem_a)
            cb = pltpu.async_copy(pb_ref.at[idx_d], b_v, sem_b)
            ca.wait()
            cb.wait()

            def edot(i, _):
                acc = jnp.zeros((16,), _f32)
                for j in range(8):
                    s = pl.ds(j * 16, 16)
                    t = jnp.maximum(a_v[i, s] + b_v[i, s], 0.0)
                    acc = acc + t * w_v[s]
                o_v[i, :] = acc
                return 0
            lax.fori_loop(0, _CK, edot, 0)
            pltpu.sync_copy(o_v, out_ref.at[pl.ds(base, _CK)])
            return 0
        lax.fori_loop(0, _NCH, chunk, 0)

    @pl.when(cid == 0)
    def _():
        run(palo, pblo, w10lo, out0)

    @pl.when(cid == 1)
    def _():
        run(pahi, pbhi, w10hi, out1)


def _sc_mesh():
    return plsc.VectorSubcoreMesh(core_axis_name="c", subcore_axis_name="s")


def _deg_call(dst, ones_h):
    return pl.kernel(
        _deg_body,
        out_type=jax.ShapeDtypeStruct((_NP, _H), _f32),
        mesh=_sc_mesh(),
        scratch_types=[pltpu.VMEM_SHARED((_NP, _H), _f32),
                       pltpu.VMEM((128, _H), _f32),
                       pltpu.VMEM((_CK, _H), _f32),
                       pltpu.VMEM((_CK,), jnp.int32)],
    )(dst, ones_h)


def _seg_call(flo, fhi, elo, ehi, src, dst):
    return pl.kernel(
        _seg_body,
        out_type=[jax.ShapeDtypeStruct((_NP, _H), _f32),
                  jax.ShapeDtypeStruct((_NP, _H), _f32)],
        mesh=_sc_mesh(),
        scratch_types=[pltpu.VMEM_SHARED((_NP, _H), _f32),
                       pltpu.VMEM((_CK, _H), _f32),
                       pltpu.VMEM((_CK, _H), _f32),
                       pltpu.VMEM((_CK,), jnp.int32),
                       pltpu.VMEM((_CK,), jnp.int32),
                       pltpu.VMEM((128, _H), _f32),
                       pltpu.SemaphoreType.DMA,
                       pltpu.SemaphoreType.DMA],
    )(flo, fhi, elo, ehi, src, dst)


def _edge_call(palo, pahi, pblo, pbhi, w10lo, w10hi, src, dst):
    return pl.kernel(
        _edge_body,
        out_type=[jax.ShapeDtypeStruct((_E, 16), _f32),
                  jax.ShapeDtypeStruct((_E, 16), _f32)],
        mesh=_sc_mesh(),
        scratch_types=[pltpu.VMEM((_CK, _H), _f32),
                       pltpu.VMEM((_CK, _H), _f32),
                       pltpu.VMEM((_H,), _f32),
                       pltpu.VMEM((_CK, 16), _f32),
                       pltpu.VMEM((_CK,), jnp.int32),
                       pltpu.VMEM((_CK,), jnp.int32),
                       pltpu.SemaphoreType.DMA,
                       pltpu.SemaphoreType.DMA],
    )(palo, pahi, pblo, pbhi, w10lo, w10hi, src, dst)

_BN = 2000
_BE = 3200


def _enc_nodes(x0, x1, wc, b1, w2t, b2):
    return pl.pallas_call(
        _enc2_body,
        grid=(_N // _BN,),
        in_specs=[_rows(_BN, 1), _rows(_BN, 1), _full((2, _D)),
                  _full((1, _D)), _full((_D, _D)), _full((1, _D))],
        out_specs=[_rows(_BN, _H), _rows(_BN, _H)],
        out_shape=[jax.ShapeDtypeStruct((_N, _H), _f32),
                   jax.ShapeDtypeStruct((_N, _H), _f32)],
    )(x0, x1, wc, b1, w2t, b2)


def _enc_edges(x0, x1, x2, wc, b1, w2t, b2):
    return pl.pallas_call(
        _enc3_body,
        grid=(_E // _BE,),
        in_specs=[_rows(_BE, 1), _rows(_BE, 1), _rows(_BE, 1),
                  _full((3, _D)), _full((1, _D)), _full((_D, _D)),
                  _full((1, _D))],
        out_specs=[_rows(_BE, _H), _rows(_BE, _H)],
        out_shape=[jax.ShapeDtypeStruct((_E, _H), _f32),
                   jax.ShapeDtypeStruct((_E, _H), _f32)],
    )(x0, x1, x2, wc, b1, w2t, b2)


def _node1(nlo, nhi, rlo, rhi, dg, swt, nwt, b1, nw2t, sw2t, b2):
    return pl.pallas_call(
        _node1_body,
        grid=(_N // _BN,),
        in_specs=[_rows(_BN, _H), _rows(_BN, _H), _rows(_BN, _H),
                  _rows(_BN, _H), _rows(_BN, _H),
                  _full((_D, _D)), _full((_D, _D)), _full((1, _D)),
                  _full((2 * _D, _D)), _full((2 * _D, _D)), _full((1, _D))],
        out_specs=[_rows(_BN, _H), _rows(_BN, _H), _rows(_BN, _D)],
        out_shape=[jax.ShapeDtypeStruct((_N, _H), _f32),
                   jax.ShapeDtypeStruct((_N, _H), _f32),
                   jax.ShapeDtypeStruct((_N, _D), _f32)],
    )(nlo, nhi, rlo, rhi, dg, swt, nwt, b1, nw2t, sw2t, b2)


def _node2(s2, rlo, rhi, dg, nlo, nhi, nw2t, sw2t, b2):
    return pl.pallas_call(
        _node2_body,
        grid=(_N // _BN,),
        in_specs=[_rows(_BN, _D), _rows(_BN, _H), _rows(_BN, _H),
                  _rows(_BN, _H), _rows(_BN, _H),
                  _rows(_BN, _H), _full((2 * _D, _D)), _full((2 * _D, _D)),
                  _full((1, _D))],
        out_specs=[_rows(_BN, _H), _rows(_BN, _H), _rows(_BN, _D)],
        out_shape=[jax.ShapeDtypeStruct((_N, _H), _f32),
                   jax.ShapeDtypeStruct((_N, _H), _f32),
                   jax.ShapeDtypeStruct((_N, _D), _f32)],
    )(s2, rlo, rhi, dg, nlo, nhi, nw2t, sw2t, b2)


def _node3(s3, rlo, rhi, dg, w9at, w9bt, b9):
    return pl.pallas_call(
        _node3_body,
        grid=(_N // _BN,),
        in_specs=[_rows(_BN, _D), _rows(_BN, _H), _rows(_BN, _H),
                  _rows(_BN, _H),
                  _full((_D, _D)), _full((_D, _D)), _full((1, _D))],
        out_specs=[_rows(_BN, _H), _rows(_BN, _H),
                   _rows(_BN, _H), _rows(_BN, _H)],
        out_shape=[jax.ShapeDtypeStruct((_N, _H), _f32),
                   jax.ShapeDtypeStruct((_N, _H), _f32),
                   jax.ShapeDtypeStruct((_N, _H), _f32),
                   jax.ShapeDtypeStruct((_N, _H), _f32)],
    )(s3, rlo, rhi, dg, w9at, w9bt, b9)


def _final(p0, p1, b10):
    return pl.pallas_call(
        _final_body,
        grid=(_E // _BE,),
        in_specs=[_rows(_BE, 16), _rows(_BE, 16), _full((1, 1))],
        out_specs=[_rows(_BE, 1)],
        out_shape=[jax.ShapeDtypeStruct((_E, 1), _f32)],
    )(p0, p1, b10)[0].reshape(_E)


def kernel(C, Fn, A, SP1, SP0, W1_w, W1_b, W2_w, W2_b, W5_w, W5_b, W6_w,
           W6_b, c1_sw, c1_nw, c1_b, c2_sw, c2_nw, c2_b, W9_w, W9_b,
           W10_w, W10_b, edge_index):
    src = edge_index[0]
    dst = edge_index[1]
    row = lambda v: v.reshape(1, -1)

    n_lo, n_hi = _enc_nodes(C, Fn, W1_w.T, row(W1_b), W2_w.T, row(W2_b))
    e_lo, e_hi = _enc_edges(A, SP1, SP0, W5_w.T, row(W5_b), W6_w.T,
                            row(W6_b))
    dg = _deg_call(dst, jnp.ones((_CK, _H), _f32))
    r1lo, r1hi = _seg_call(n_lo, n_hi, e_lo, e_hi, src, dst)
    m2lo, m2hi, s2 = _node1(n_lo, n_hi, r1lo, r1hi, dg,
                            c1_sw.T, c1_nw.T, row(c1_b),
                            c2_nw.T, c2_sw.T, row(c2_b))
    r2lo, r2hi = _seg_call(m2lo, m2hi, e_lo, e_hi, src, dst)
    m3lo, m3hi, s3 = _node2(s2, r2lo, r2hi, dg, n_lo, n_hi,
                            c2_nw.T, c2_sw.T, row(c2_b))
    r3lo, r3hi = _seg_call(m3lo, m3hi, e_lo, e_hi, src, dst)
    palo, pahi, pblo, pbhi = _node3(s3, r3lo, r3hi, dg,
                                    W9_w[:, :_D].T, W9_w[:, _D:].T,
                                    row(W9_b))
    w10 = W10_w[0]
    p0, p1 = _edge_call(palo, pahi, pblo, pbhi, w10[:_H], w10[_H:],
                        src, dst)
    return _final(p0, p1, W10_b.reshape(1, 1))


# speedup vs baseline: 2.7340x; 1.3784x over previous
import functools

import jax
import jax.numpy as jnp
from jax import lax
from jax.experimental import pallas as pl
from jax.experimental.pallas import tpu as pltpu
from jax.experimental.pallas import tpu_sc as plsc

_N = 10000
_E = 160000
_D = 256
_H = 128
_NP = 10240
_RPS = 640
_NSUB = 16
_EPS = _E // _NSUB
_CK = 80
_NCH = _EPS // _CK
_EPW = _E // 32
_CKD = 40
_NCHD = _EPW // _CKD

_f32 = jnp.float32


def _enc2_body(x0, x1, wc, b1, w2t, b2, lo, hi):
    h = x0[...] * wc[0:1, :] + x1[...] * wc[1:2, :] + b1[...]
    h = jnp.maximum(h, 0.0)
    o = jnp.dot(h, w2t[...], preferred_element_type=_f32) + b2[...]
    lo[...] = o[:, :_H]
    hi[...] = o[:, _H:]


def _enc3_body(x0, x1, x2, wc, b1, w2t, b2, lo, hi):
    h = (x0[...] * wc[0:1, :] + x1[...] * wc[1:2, :] + x2[...] * wc[2:3, :]
         + b1[...])
    h = jnp.maximum(h, 0.0)
    o = jnp.dot(h, w2t[...], preferred_element_type=_f32) + b2[...]
    lo[...] = o[:, :_H]
    hi[...] = o[:, _H:]


def _node1_body(nlo, nhi, rlo, rhi, dg, swt, nwt, b1, nw2t, sw2t, b2,
                m2lo, m2hi, s2):
    n = jnp.concatenate([nlo[...], nhi[...]], axis=1)
    raw = jnp.concatenate([rlo[...], rhi[...]], axis=1)
    deg = jnp.maximum(dg[:, :1], 1.0)
    neigh = jnp.dot(raw / deg, nwt[...], preferred_element_type=_f32)
    h1 = jnp.maximum(
        jnp.dot(n, swt[...], preferred_element_type=_f32) + neigh + b1[...],
        0.0)
    hcat = jnp.concatenate([h1, n], axis=1)
    m2 = jnp.dot(hcat, nw2t[...], preferred_element_type=_f32)
    m2lo[...] = m2[:, :_H]
    m2hi[...] = m2[:, _H:]
    s2[...] = jnp.dot(hcat, sw2t[...], preferred_element_type=_f32) + b2[...]


def _node2_body(s2, rlo, rhi, dg, nlo, nhi, nw2t, sw2t, b2,
                m3lo, m3hi, s3):
    raw = jnp.concatenate([rlo[...], rhi[...]], axis=1)
    deg = jnp.maximum(dg[:, :1], 1.0)
    h2 = jnp.maximum(s2[...] + raw / deg, 0.0)
    n = jnp.concatenate([nlo[...], nhi[...]], axis=1)
    hcat = jnp.concatenate([h2, n], axis=1)
    m3 = jnp.dot(hcat, nw2t[...], preferred_element_type=_f32)
    m3lo[...] = m3[:, :_H]
    m3hi[...] = m3[:, _H:]
    s3[...] = jnp.dot(hcat, sw2t[...], preferred_element_type=_f32) + b2[...]


def _node3_body(s3, rlo, rhi, dg, w9at, w9bt, b9,
                palo, pahi, pblo, pbhi):
    raw = jnp.concatenate([rlo[...], rhi[...]], axis=1)
    deg = jnp.maximum(dg[:, :1], 1.0)
    h3 = s3[...] + raw / deg
    pa = jnp.dot(h3, w9at[...], preferred_element_type=_f32) + b9[...]
    pb = jnp.dot(h3, w9bt[...], preferred_element_type=_f32)
    palo[...] = pa[:, :_H]
    pahi[...] = pa[:, _H:]
    pblo[...] = pb[:, :_H]
    pbhi[...] = pb[:, _H:]


def _final_body(p0, p1, b10, o):
    o[...] = jnp.sum(p0[...] + p1[...], axis=1, keepdims=True) + b10[0, 0]


def _full(shape):
    return pl.BlockSpec(shape, lambda i: tuple(0 for _ in shape))


def _rows(bs, w):
    return pl.BlockSpec((bs, w), lambda i: (i, 0))


def _deg_body(dst, ones_h, out0, acc, zvm, ones_v, idx_d):
    cid = lax.axis_index("c")
    sid = lax.axis_index("s")

    def zrow(i, _):
        for j in range(8):
            zvm[i, pl.ds(j * 16, 16)] = jnp.zeros((16,), _f32)
        return 0
    lax.fori_loop(0, 128, zrow, 0)
    pltpu.sync_copy(ones_h, ones_v)

    def zacc(k, _):
        pltpu.sync_copy(zvm, acc.at[pl.ds(sid * _RPS + k * 128, 128)])
        return 0
    lax.fori_loop(0, 5, zacc, 0)
    plsc.subcore_barrier()

    def chunk(ch, _):
        base = sid * _EPS + ch * _CK
        pltpu.sync_copy(dst.at[pl.ds(base, _CK)], idx_d)
        pltpu.sync_copy(ones_v, acc.at[idx_d], add=True)
        return 0
    lax.fori_loop(0, _NCH, chunk, 0)
    plsc.subcore_barrier()

    @pl.when(cid == 0)
    def _():
        pltpu.sync_copy(acc.at[pl.ds(sid * _RPS, _RPS)],
                        out0.at[pl.ds(sid * _RPS, _RPS)])


def _seg_body(flo, fhi, elo, ehi, src, dst, out_lo, out_hi,
              acc, rows_v, e_v, idx_s, idx_d, sem_g, sem_e,
              rows_v2, e_v2, idx_s2, idx_d2, sem_g2, sem_e2):
    cid = lax.axis_index("c")
    sid = lax.axis_index("s")

    def zrow(i, _):
        for j in range(8):
            rows_v[i, pl.ds(j * 16, 16)] = jnp.zeros((16,), _f32)
        return 0
    lax.fori_loop(0, _CK, zrow, 0)

    def zacc(k, _):
        pltpu.sync_copy(rows_v, acc.at[pl.ds(sid * _RPS + k * _CK, _CK)])
        return 0
    lax.fori_loop(0, _RPS // _CK, zacc, 0)
    plsc.subcore_barrier()

    def run(feat_ref, e_ref):
        bufs = ((rows_v, e_v, idx_s, idx_d, sem_g, sem_e),
                (rows_v2, e_v2, idx_s2, idx_d2, sem_g2, sem_e2))

        def start(ch, b):
            rows, ev, ixs, ixd, sg, se = bufs[b]
            base = sid * _EPS + ch * _CK
            pltpu.sync_copy(src.at[pl.ds(base, _CK)], ixs)
            pltpu.sync_copy(dst.at[pl.ds(base, _CK)], ixd)
            pltpu.async_copy(feat_ref.at[ixs], rows, sg)
            pltpu.async_copy(e_ref.at[pl.ds(base, _CK)], ev, se)

        def finish(ch, b):
            rows, ev, ixs, ixd, sg, se = bufs[b]
            base = sid * _EPS + ch * _CK
            pltpu.make_async_copy(feat_ref.at[ixs], rows, sg).wait()
            pltpu.make_async_copy(e_ref.at[pl.ds(base, _CK)], ev, se).wait()

            def mul(i, _):
                for j in range(8):
                    s = pl.ds(j * 16, 16)
                    rows[i, s] = rows[i, s] * ev[i, s]
                return 0
            lax.fori_loop(0, _CK, mul, 0)
            pltpu.sync_copy(rows, acc.at[ixd], add=True)

        start(0, 0)

        def pair(k, _):
            ch = 2 * k
            start(ch + 1, 1)
            finish(ch, 0)
            start(ch + 2, 0)
            finish(ch + 1, 1)
            return 0
        lax.fori_loop(0, (_NCH - 1) // 2, pair, 0)
        finish(_NCH - 1, 0)

    @pl.when(cid == 0)
    def _():
        run(flo, elo)

    @pl.when(cid == 1)
    def _():
        run(fhi, ehi)

    plsc.subcore_barrier()

    @pl.when(cid == 0)
    def _():
        pltpu.sync_copy(acc.at[pl.ds(sid * _RPS, _RPS)],
                        out_lo.at[pl.ds(sid * _RPS, _RPS)])

    @pl.when(cid == 1)
    def _():
        pltpu.sync_copy(acc.at[pl.ds(sid * _RPS, _RPS)],
                        out_hi.at[pl.ds(sid * _RPS, _RPS)])


def _edge_body(palo, pahi, pblo, pbhi, w10lo, w10hi, src, dst, out0, out1,
               a_v, b_v, w_v, o_v, idx_s, idx_d, sem_a, sem_b,
               a_v2, b_v2, idx_s2, idx_d2, sem_a2, sem_b2):
    cid = lax.axis_index("c")
    sid = lax.axis_index("s")

    def run(pa_ref, pb_ref, w_ref, out_ref):
        pltpu.sync_copy(w_ref, w_v)
        bufs = ((a_v, b_v, idx_s, idx_d, sem_a, sem_b),
                (a_v2, b_v2, idx_s2, idx_d2, sem_a2, sem_b2))

        def start(ch, b):
            av, bv, ixs, ixd, sa, sb = bufs[b]
            base = sid * _EPS + ch * _CK
            pltpu.sync_copy(src.at[pl.ds(base, _CK)], ixs)
            pltpu.sync_copy(dst.at[pl.ds(base, _CK)], ixd)
            pltpu.async_copy(pa_ref.at[ixs], av, sa)
            pltpu.async_copy(pb_ref.at[ixd], bv, sb)

        def finish(ch, b):
            av, bv, ixs, ixd, sa, sb = bufs[b]
            base = sid * _EPS + ch * _CK
            pltpu.make_async_copy(pa_ref.at[ixs], av, sa).wait()
            pltpu.make_async_copy(pb_ref.at[ixd], bv, sb).wait()

            def edot(i, _):
                acc = jnp.zeros((16,), _f32)
                for j in range(8):
                    s = pl.ds(j * 16, 16)
                    t = jnp.maximum(av[i, s] + bv[i, s], 0.0)
                    acc = acc + t * w_v[s]
                o_v[i, :] = acc
                return 0
            lax.fori_loop(0, _CK, edot, 0)
            pltpu.sync_copy(o_v, out_ref.at[pl.ds(base, _CK)])

        start(0, 0)

        def pair(k, _):
            ch = 2 * k
            start(ch + 1, 1)
            finish(ch, 0)
            start(ch + 2, 0)
            finish(ch + 1, 1)
            return 0
        lax.fori_loop(0, (_NCH - 1) // 2, pair, 0)
        finish(_NCH - 1, 0)

    @pl.when(cid == 0)
    def _():
        run(palo, pblo, w10lo, out0)

    @pl.when(cid == 1)
    def _():
        run(pahi, pbhi, w10hi, out1)


def _sc_mesh():
    return plsc.VectorSubcoreMesh(core_axis_name="c", subcore_axis_name="s")


def _deg_call(dst, ones_h):
    return pl.kernel(
        _deg_body,
        out_type=jax.ShapeDtypeStruct((_NP, _H), _f32),
        mesh=_sc_mesh(),
        scratch_types=[pltpu.VMEM_SHARED((_NP, _H), _f32),
                       pltpu.VMEM((128, _H), _f32),
                       pltpu.VMEM((_CK, _H), _f32),
                       pltpu.VMEM((_CK,), jnp.int32)],
    )(dst, ones_h)


def _seg_call(flo, fhi, elo, ehi, src, dst):
    return pl.kernel(
        _seg_body,
        out_type=[jax.ShapeDtypeStruct((_NP, _H), _f32),
                  jax.ShapeDtypeStruct((_NP, _H), _f32)],
        mesh=_sc_mesh(),
        scratch_types=[pltpu.VMEM_SHARED((_NP, _H), _f32),
                       pltpu.VMEM((_CK, _H), _f32),
                       pltpu.VMEM((_CK, _H), _f32),
                       pltpu.VMEM((_CK,), jnp.int32),
                       pltpu.VMEM((_CK,), jnp.int32),
                       pltpu.SemaphoreType.DMA,
                       pltpu.SemaphoreType.DMA,
                       pltpu.VMEM((_CK, _H), _f32),
                       pltpu.VMEM((_CK, _H), _f32),
                       pltpu.VMEM((_CK,), jnp.int32),
                       pltpu.VMEM((_CK,), jnp.int32),
                       pltpu.SemaphoreType.DMA,
                       pltpu.SemaphoreType.DMA],
    )(flo, fhi, elo, ehi, src, dst)


def _edge_call(palo, pahi, pblo, pbhi, w10lo, w10hi, src, dst):
    return pl.kernel(
        _edge_body,
        out_type=[jax.ShapeDtypeStruct((_E, 16), _f32),
                  jax.ShapeDtypeStruct((_E, 16), _f32)],
        mesh=_sc_mesh(),
        scratch_types=[pltpu.VMEM((_CK, _H), _f32),
                       pltpu.VMEM((_CK, _H), _f32),
                       pltpu.VMEM((_H,), _f32),
                       pltpu.VMEM((_CK, 16), _f32),
                       pltpu.VMEM((_CK,), jnp.int32),
                       pltpu.VMEM((_CK,), jnp.int32),
                       pltpu.SemaphoreType.DMA,
                       pltpu.SemaphoreType.DMA,
                       pltpu.VMEM((_CK, _H), _f32),
                       pltpu.VMEM((_CK, _H), _f32),
                       pltpu.VMEM((_CK,), jnp.int32),
                       pltpu.VMEM((_CK,), jnp.int32),
                       pltpu.SemaphoreType.DMA,
                       pltpu.SemaphoreType.DMA],
    )(palo, pahi, pblo, pbhi, w10lo, w10hi, src, dst)

_BN = 2000
_BE = 3200


def _enc_nodes(x0, x1, wc, b1, w2t, b2):
    return pl.pallas_call(
        _enc2_body,
        grid=(_N // _BN,),
        in_specs=[_rows(_BN, 1), _rows(_BN, 1), _full((2, _D)),
                  _full((1, _D)), _full((_D, _D)), _full((1, _D))],
        out_specs=[_rows(_BN, _H), _rows(_BN, _H)],
        out_shape=[jax.ShapeDtypeStruct((_N, _H), _f32),
                   jax.ShapeDtypeStruct((_N, _H), _f32)],
    )(x0, x1, wc, b1, w2t, b2)


def _enc_edges(x0, x1, x2, wc, b1, w2t, b2):
    return pl.pallas_call(
        _enc3_body,
        grid=(_E // _BE,),
        in_specs=[_rows(_BE, 1), _rows(_BE, 1), _rows(_BE, 1),
                  _full((3, _D)), _full((1, _D)), _full((_D, _D)),
                  _full((1, _D))],
        out_specs=[_rows(_BE, _H), _rows(_BE, _H)],
        out_shape=[jax.ShapeDtypeStruct((_E, _H), _f32),
                   jax.ShapeDtypeStruct((_E, _H), _f32)],
    )(x0, x1, x2, wc, b1, w2t, b2)


def _node1(nlo, nhi, rlo, rhi, dg, swt, nwt, b1, nw2t, sw2t, b2):
    return pl.pallas_call(
        _node1_body,
        grid=(_N // _BN,),
        in_specs=[_rows(_BN, _H), _rows(_BN, _H), _rows(_BN, _H),
                  _rows(_BN, _H), _rows(_BN, _H),
                  _full((_D, _D)), _full((_D, _D)), _full((1, _D)),
                  _full((2 * _D, _D)), _full((2 * _D, _D)), _full((1, _D))],
        out_specs=[_rows(_BN, _H), _rows(_BN, _H), _rows(_BN, _D)],
        out_shape=[jax.ShapeDtypeStruct((_N, _H), _f32),
                   jax.ShapeDtypeStruct((_N, _H), _f32),
                   jax.ShapeDtypeStruct((_N, _D), _f32)],
    )(nlo, nhi, rlo, rhi, dg, swt, nwt, b1, nw2t, sw2t, b2)


def _node2(s2, rlo, rhi, dg, nlo, nhi, nw2t, sw2t, b2):
    return pl.pallas_call(
        _node2_body,
        grid=(_N // _BN,),
        in_specs=[_rows(_BN, _D), _rows(_BN, _H), _rows(_BN, _H),
                  _rows(_BN, _H), _rows(_BN, _H),
                  _rows(_BN, _H), _full((2 * _D, _D)), _full((2 * _D, _D)),
                  _full((1, _D))],
        out_specs=[_rows(_BN, _H), _rows(_BN, _H), _rows(_BN, _D)],
        out_shape=[jax.ShapeDtypeStruct((_N, _H), _f32),
                   jax.ShapeDtypeStruct((_N, _H), _f32),
                   jax.ShapeDtypeStruct((_N, _D), _f32)],
    )(s2, rlo, rhi, dg, nlo, nhi, nw2t, sw2t, b2)


def _node3(s3, rlo, rhi, dg, w9at, w9bt, b9):
    return pl.pallas_call(
        _node3_body,
        grid=(_N // _BN,),
        in_specs=[_rows(_BN, _D), _rows(_BN, _H), _rows(_BN, _H),
                  _rows(_BN, _H),
                  _full((_D, _D)), _full((_D, _D)), _full((1, _D))],
        out_specs=[_rows(_BN, _H), _rows(_BN, _H),
                   _rows(_BN, _H), _rows(_BN, _H)],
        out_shape=[jax.ShapeDtypeStruct((_N, _H), _f32),
                   jax.ShapeDtypeStruct((_N, _H), _f32),
                   jax.ShapeDtypeStruct((_N, _H), _f32),
                   jax.ShapeDtypeStruct((_N, _H), _f32)],
    )(s3, rlo, rhi, dg, w9at, w9bt, b9)


def _final(p0, p1, b10):
    return pl.pallas_call(
        _final_body,
        grid=(_E // _BE,),
        in_specs=[_rows(_BE, 16), _rows(_BE, 16), _full((1, 1))],
        out_specs=[_rows(_BE, 1)],
        out_shape=[jax.ShapeDtypeStruct((_E, 1), _f32)],
    )(p0, p1, b10)[0].reshape(_E)


def kernel(C, Fn, A, SP1, SP0, W1_w, W1_b, W2_w, W2_b, W5_w, W5_b, W6_w,
           W6_b, c1_sw, c1_nw, c1_b, c2_sw, c2_nw, c2_b, W9_w, W9_b,
           W10_w, W10_b, edge_index):
    src = edge_index[0]
    dst = edge_index[1]
    row = lambda v: v.reshape(1, -1)

    n_lo, n_hi = _enc_nodes(C, Fn, W1_w.T, row(W1_b), W2_w.T, row(W2_b))
    e_lo, e_hi = _enc_edges(A, SP1, SP0, W5_w.T, row(W5_b), W6_w.T,
                            row(W6_b))
    dg = _deg_call(dst, jnp.ones((_CK, _H), _f32))
    r1lo, r1hi = _seg_call(n_lo, n_hi, e_lo, e_hi, src, dst)
    m2lo, m2hi, s2 = _node1(n_lo, n_hi, r1lo, r1hi, dg,
                            c1_sw.T, c1_nw.T, row(c1_b),
                            c2_nw.T, c2_sw.T, row(c2_b))
    r2lo, r2hi = _seg_call(m2lo, m2hi, e_lo, e_hi, src, dst)
    m3lo, m3hi, s3 = _node2(s2, r2lo, r2hi, dg, n_lo, n_hi,
                            c2_nw.T, c2_sw.T, row(c2_b))
    r3lo, r3hi = _seg_call(m3lo, m3hi, e_lo, e_hi, src, dst)
    palo, pahi, pblo, pbhi = _node3(s3, r3lo, r3hi, dg,
                                    W9_w[:, :_D].T, W9_w[:, _D:].T,
                                    row(W9_b))
    w10 = W10_w[0]
    p0, p1 = _edge_call(palo, pahi, pblo, pbhi, w10[:_H], w10[_H:],
                        src, dst)
    return _final(p0, p1, W10_b.reshape(1, 1))


# speedup vs baseline: 3.0816x; 1.1272x over previous
import functools

import jax
import jax.numpy as jnp
from jax import lax
from jax.experimental import pallas as pl
from jax.experimental.pallas import tpu as pltpu
from jax.experimental.pallas import tpu_sc as plsc

_N = 10000
_E = 160000
_D = 256
_H = 128
_NP = 10240
_RPS = 640
_NSUB = 16
_EPS = _E // _NSUB
_CK = 80
_NCH = _EPS // _CK
_EPW = _E // 32
_CKD = 40
_NCHD = _EPW // _CKD

_f32 = jnp.float32


def _enc2_body(x0, x1, wc, b1, w2t, b2, lo, hi):
    h = x0[...] * wc[0:1, :] + x1[...] * wc[1:2, :] + b1[...]
    h = jnp.maximum(h, 0.0)
    o = jnp.dot(h, w2t[...], preferred_element_type=_f32) + b2[...]
    lo[...] = o[:, :_H]
    hi[...] = o[:, _H:]


def _enc3_body(x0, x1, x2, wc, b1, w2t, b2, lo, hi):
    h = (x0[...] * wc[0:1, :] + x1[...] * wc[1:2, :] + x2[...] * wc[2:3, :]
         + b1[...])
    h = jnp.maximum(h, 0.0)
    o = jnp.dot(h, w2t[...], preferred_element_type=_f32) + b2[...]
    lo[...] = o[:, :_H]
    hi[...] = o[:, _H:]


def _node1_body(nlo, nhi, rlo, rhi, dg, swt, nwt, b1, nw2t, sw2t, b2,
                m2lo, m2hi, s2):
    n = jnp.concatenate([nlo[...], nhi[...]], axis=1)
    raw = jnp.concatenate([rlo[...], rhi[...]], axis=1)
    deg = jnp.maximum(dg[:, :1], 1.0)
    neigh = jnp.dot(raw / deg, nwt[...], preferred_element_type=_f32)
    h1 = jnp.maximum(
        jnp.dot(n, swt[...], preferred_element_type=_f32) + neigh + b1[...],
        0.0)
    hcat = jnp.concatenate([h1, n], axis=1)
    m2 = jnp.dot(hcat, nw2t[...], preferred_element_type=_f32)
    m2lo[...] = m2[:, :_H]
    m2hi[...] = m2[:, _H:]
    s2[...] = jnp.dot(hcat, sw2t[...], preferred_element_type=_f32) + b2[...]


def _node2_body(s2, rlo, rhi, dg, nlo, nhi, nw2t, sw2t, b2,
                m3lo, m3hi, s3):
    raw = jnp.concatenate([rlo[...], rhi[...]], axis=1)
    deg = jnp.maximum(dg[:, :1], 1.0)
    h2 = jnp.maximum(s2[...] + raw / deg, 0.0)
    n = jnp.concatenate([nlo[...], nhi[...]], axis=1)
    hcat = jnp.concatenate([h2, n], axis=1)
    m3 = jnp.dot(hcat, nw2t[...], preferred_element_type=_f32)
    m3lo[...] = m3[:, :_H]
    m3hi[...] = m3[:, _H:]
    s3[...] = jnp.dot(hcat, sw2t[...], preferred_element_type=_f32) + b2[...]


def _node3_body(s3, rlo, rhi, dg, w9at, w9bt, b9,
                palo, pahi, pblo, pbhi):
    raw = jnp.concatenate([rlo[...], rhi[...]], axis=1)
    deg = jnp.maximum(dg[:, :1], 1.0)
    h3 = s3[...] + raw / deg
    pa = jnp.dot(h3, w9at[...], preferred_element_type=_f32) + b9[...]
    pb = jnp.dot(h3, w9bt[...], preferred_element_type=_f32)
    palo[...] = pa[:, :_H]
    pahi[...] = pa[:, _H:]
    pblo[...] = pb[:, :_H]
    pbhi[...] = pb[:, _H:]


def _final_body(p0, p1, b10, o):
    o[...] = jnp.sum(p0[...] + p1[...], axis=1, keepdims=True) + b10[0, 0]


def _full(shape):
    return pl.BlockSpec(shape, lambda i: tuple(0 for _ in shape))


def _rows(bs, w):
    return pl.BlockSpec((bs, w), lambda i: (i, 0))


def _deg_body(dst, ones_h, out0, acc, zvm, ones_v, idx_d):
    cid = lax.axis_index("c")
    sid = lax.axis_index("s")

    def zrow(i, _):
        for j in range(8):
            zvm[i, pl.ds(j * 16, 16)] = jnp.zeros((16,), _f32)
        return 0
    lax.fori_loop(0, 128, zrow, 0)
    pltpu.sync_copy(ones_h, ones_v)

    def zacc(k, _):
        pltpu.sync_copy(zvm, acc.at[pl.ds(sid * _RPS + k * 128, 128)])
        return 0
    lax.fori_loop(0, 5, zacc, 0)
    plsc.subcore_barrier()

    def chunk(ch, _):
        base = sid * _EPS + ch * _CK
        pltpu.sync_copy(dst.at[pl.ds(base, _CK)], idx_d)
        pltpu.sync_copy(ones_v, acc.at[idx_d], add=True)
        return 0
    lax.fori_loop(0, _NCH, chunk, 0)
    plsc.subcore_barrier()

    @pl.when(cid == 0)
    def _():
        pltpu.sync_copy(acc.at[pl.ds(sid * _RPS, _RPS)],
                        out0.at[pl.ds(sid * _RPS, _RPS)])


def _seg_body(flo, fhi, elo, ehi, idxp, out_lo, out_hi,
              acc, rows_v, e_v, ix2, sem_g, sem_e,
              rows_v2, e_v2, ix2b, sem_g2, sem_e2):
    cid = lax.axis_index("c")
    sid = lax.axis_index("s")

    def zrow(i, _):
        for j in range(8):
            rows_v[i, pl.ds(j * 16, 16)] = jnp.zeros((16,), _f32)
        return 0
    lax.fori_loop(0, _CK, zrow, 0)

    def zacc(k, _):
        pltpu.sync_copy(rows_v, acc.at[pl.ds(sid * _RPS + k * _CK, _CK)])
        return 0
    lax.fori_loop(0, _RPS // _CK, zacc, 0)
    plsc.subcore_barrier()

    def run(feat_ref, e_ref):
        bufs = ((rows_v, e_v, ix2, sem_g, sem_e),
                (rows_v2, e_v2, ix2b, sem_g2, sem_e2))

        def start(ch, b):
            rows, ev, ix, sg, se = bufs[b]
            base = sid * _EPS + ch * _CK
            pltpu.sync_copy(idxp.at[sid * _NCH + ch], ix)
            pltpu.async_copy(feat_ref.at[ix.at[0]], rows, sg)
            pltpu.async_copy(e_ref.at[pl.ds(base, _CK)], ev, se)

        def finish(ch, b):
            rows, ev, ix, sg, se = bufs[b]
            base = sid * _EPS + ch * _CK
            pltpu.make_async_copy(feat_ref.at[ix.at[0]], rows, sg).wait()
            pltpu.make_async_copy(e_ref.at[pl.ds(base, _CK)], ev, se).wait()

            @plsc.parallel_loop(0, _CK, unroll=2)
            def _mul(i):
                for j in range(8):
                    s = pl.ds(j * 16, 16)
                    rows[i, s] = rows[i, s] * ev[i, s]
            pltpu.sync_copy(rows, acc.at[ix.at[1]], add=True)

        start(0, 0)

        def pair(k, _):
            ch = 2 * k
            start(ch + 1, 1)
            finish(ch, 0)
            start(ch + 2, 0)
            finish(ch + 1, 1)
            return 0
        lax.fori_loop(0, (_NCH - 1) // 2, pair, 0)
        finish(_NCH - 1, 0)

    @pl.when(cid == 0)
    def _():
        run(flo, elo)

    @pl.when(cid == 1)
    def _():
        run(fhi, ehi)

    plsc.subcore_barrier()

    @pl.when(cid == 0)
    def _():
        pltpu.sync_copy(acc.at[pl.ds(sid * _RPS, _RPS)],
                        out_lo.at[pl.ds(sid * _RPS, _RPS)])

    @pl.when(cid == 1)
    def _():
        pltpu.sync_copy(acc.at[pl.ds(sid * _RPS, _RPS)],
                        out_hi.at[pl.ds(sid * _RPS, _RPS)])


def _edge_body(palo, pahi, pblo, pbhi, w10lo, w10hi, idxp, out0, out1,
               a_v, b_v, w_v, o_v, ix2, sem_a, sem_b,
               a_v2, b_v2, ix2b, sem_a2, sem_b2):
    cid = lax.axis_index("c")
    sid = lax.axis_index("s")

    def run(pa_ref, pb_ref, w_ref, out_ref):
        pltpu.sync_copy(w_ref, w_v)
        bufs = ((a_v, b_v, ix2, sem_a, sem_b),
                (a_v2, b_v2, ix2b, sem_a2, sem_b2))

        def start(ch, b):
            av, bv, ix, sa, sb = bufs[b]
            pltpu.sync_copy(idxp.at[sid * _NCH + ch], ix)
            pltpu.async_copy(pa_ref.at[ix.at[0]], av, sa)
            pltpu.async_copy(pb_ref.at[ix.at[1]], bv, sb)

        def finish(ch, b):
            av, bv, ix, sa, sb = bufs[b]
            base = sid * _EPS + ch * _CK
            pltpu.make_async_copy(pa_ref.at[ix.at[0]], av, sa).wait()
            pltpu.make_async_copy(pb_ref.at[ix.at[1]], bv, sb).wait()

            @plsc.parallel_loop(0, _CK, unroll=2)
            def _edot(i):
                acc = jnp.zeros((16,), _f32)
                for j in range(8):
                    s = pl.ds(j * 16, 16)
                    t = jnp.maximum(av[i, s] + bv[i, s], 0.0)
                    acc = acc + t * w_v[s]
                o_v[i, :] = acc
            pltpu.sync_copy(o_v, out_ref.at[pl.ds(base, _CK)])

        start(0, 0)

        def pair(k, _):
            ch = 2 * k
            start(ch + 1, 1)
            finish(ch, 0)
            start(ch + 2, 0)
            finish(ch + 1, 1)
            return 0
        lax.fori_loop(0, (_NCH - 1) // 2, pair, 0)
        finish(_NCH - 1, 0)

    @pl.when(cid == 0)
    def _():
        run(palo, pblo, w10lo, out0)

    @pl.when(cid == 1)
    def _():
        run(pahi, pbhi, w10hi, out1)


def _sc_mesh():
    return plsc.VectorSubcoreMesh(core_axis_name="c", subcore_axis_name="s")


def _deg_call(dst, ones_h):
    return pl.kernel(
        _deg_body,
        out_type=jax.ShapeDtypeStruct((_NP, _H), _f32),
        mesh=_sc_mesh(),
        scratch_types=[pltpu.VMEM_SHARED((_NP, _H), _f32),
                       pltpu.VMEM((128, _H), _f32),
                       pltpu.VMEM((_CK, _H), _f32),
                       pltpu.VMEM((_CK,), jnp.int32)],
    )(dst, ones_h)


def _seg_call(flo, fhi, elo, ehi, idxp):
    return pl.kernel(
        _seg_body,
        out_type=[jax.ShapeDtypeStruct((_NP, _H), _f32),
                  jax.ShapeDtypeStruct((_NP, _H), _f32)],
        mesh=_sc_mesh(),
        scratch_types=[pltpu.VMEM_SHARED((_NP, _H), _f32),
                       pltpu.VMEM((_CK, _H), _f32),
                       pltpu.VMEM((_CK, _H), _f32),
                       pltpu.VMEM((2, _CK), jnp.int32),
                       pltpu.SemaphoreType.DMA,
                       pltpu.SemaphoreType.DMA,
                       pltpu.VMEM((_CK, _H), _f32),
                       pltpu.VMEM((_CK, _H), _f32),
                       pltpu.VMEM((2, _CK), jnp.int32),
                       pltpu.SemaphoreType.DMA,
                       pltpu.SemaphoreType.DMA],
    )(flo, fhi, elo, ehi, idxp)


def _edge_call(palo, pahi, pblo, pbhi, w10lo, w10hi, idxp):
    return pl.kernel(
        _edge_body,
        out_type=[jax.ShapeDtypeStruct((_E, 16), _f32),
                  jax.ShapeDtypeStruct((_E, 16), _f32)],
        mesh=_sc_mesh(),
        scratch_types=[pltpu.VMEM((_CK, _H), _f32),
                       pltpu.VMEM((_CK, _H), _f32),
                       pltpu.VMEM((_H,), _f32),
                       pltpu.VMEM((_CK, 16), _f32),
                       pltpu.VMEM((2, _CK), jnp.int32),
                       pltpu.SemaphoreType.DMA,
                       pltpu.SemaphoreType.DMA,
                       pltpu.VMEM((_CK, _H), _f32),
                       pltpu.VMEM((_CK, _H), _f32),
                       pltpu.VMEM((2, _CK), jnp.int32),
                       pltpu.SemaphoreType.DMA,
                       pltpu.SemaphoreType.DMA],
    )(palo, pahi, pblo, pbhi, w10lo, w10hi, idxp)

_BN = 2000
_BE = 3200


def _enc_nodes(x0, x1, wc, b1, w2t, b2):
    return pl.pallas_call(
        _enc2_body,
        grid=(_N // _BN,),
        in_specs=[_rows(_BN, 1), _rows(_BN, 1), _full((2, _D)),
                  _full((1, _D)), _full((_D, _D)), _full((1, _D))],
        out_specs=[_rows(_BN, _H), _rows(_BN, _H)],
        out_shape=[jax.ShapeDtypeStruct((_N, _H), _f32),
                   jax.ShapeDtypeStruct((_N, _H), _f32)],
    )(x0, x1, wc, b1, w2t, b2)


def _enc_edges(x0, x1, x2, wc, b1, w2t, b2):
    return pl.pallas_call(
        _enc3_body,
        grid=(_E // _BE,),
        in_specs=[_rows(_BE, 1), _rows(_BE, 1), _rows(_BE, 1),
                  _full((3, _D)), _full((1, _D)), _full((_D, _D)),
                  _full((1, _D))],
        out_specs=[_rows(_BE, _H), _rows(_BE, _H)],
        out_shape=[jax.ShapeDtypeStruct((_E, _H), _f32),
                   jax.ShapeDtypeStruct((_E, _H), _f32)],
    )(x0, x1, x2, wc, b1, w2t, b2)


def _node1(nlo, nhi, rlo, rhi, dg, swt, nwt, b1, nw2t, sw2t, b2):
    return pl.pallas_call(
        _node1_body,
        grid=(_N // _BN,),
        in_specs=[_rows(_BN, _H), _rows(_BN, _H), _rows(_BN, _H),
                  _rows(_BN, _H), _rows(_BN, _H),
                  _full((_D, _D)), _full((_D, _D)), _full((1, _D)),
                  _full((2 * _D, _D)), _full((2 * _D, _D)), _full((1, _D))],
        out_specs=[_rows(_BN, _H), _rows(_BN, _H), _rows(_BN, _D)],
        out_shape=[jax.ShapeDtypeStruct((_N, _H), _f32),
                   jax.ShapeDtypeStruct((_N, _H), _f32),
                   jax.ShapeDtypeStruct((_N, _D), _f32)],
    )(nlo, nhi, rlo, rhi, dg, swt, nwt, b1, nw2t, sw2t, b2)


def _node2(s2, rlo, rhi, dg, nlo, nhi, nw2t, sw2t, b2):
    return pl.pallas_call(
        _node2_body,
        grid=(_N // _BN,),
        in_specs=[_rows(_BN, _D), _rows(_BN, _H), _rows(_BN, _H),
                  _rows(_BN, _H), _rows(_BN, _H),
                  _rows(_BN, _H), _full((2 * _D, _D)), _full((2 * _D, _D)),
                  _full((1, _D))],
        out_specs=[_rows(_BN, _H), _rows(_BN, _H), _rows(_BN, _D)],
        out_shape=[jax.ShapeDtypeStruct((_N, _H), _f32),
                   jax.ShapeDtypeStruct((_N, _H), _f32),
                   jax.ShapeDtypeStruct((_N, _D), _f32)],
    )(s2, rlo, rhi, dg, nlo, nhi, nw2t, sw2t, b2)


def _node3(s3, rlo, rhi, dg, w9at, w9bt, b9):
    return pl.pallas_call(
        _node3_body,
        grid=(_N // _BN,),
        in_specs=[_rows(_BN, _D), _rows(_BN, _H), _rows(_BN, _H),
                  _rows(_BN, _H),
                  _full((_D, _D)), _full((_D, _D)), _full((1, _D))],
        out_specs=[_rows(_BN, _H), _rows(_BN, _H),
                   _rows(_BN, _H), _rows(_BN, _H)],
        out_shape=[jax.ShapeDtypeStruct((_N, _H), _f32),
                   jax.ShapeDtypeStruct((_N, _H), _f32),
                   jax.ShapeDtypeStruct((_N, _H), _f32),
                   jax.ShapeDtypeStruct((_N, _H), _f32)],
    )(s3, rlo, rhi, dg, w9at, w9bt, b9)


def _final(p0, p1, b10):
    return pl.pallas_call(
        _final_body,
        grid=(_E // _BE,),
        in_specs=[_rows(_BE, 16), _rows(_BE, 16), _full((1, 1))],
        out_specs=[_rows(_BE, 1)],
        out_shape=[jax.ShapeDtypeStruct((_E, 1), _f32)],
    )(p0, p1, b10)[0].reshape(_E)


def kernel(C, Fn, A, SP1, SP0, W1_w, W1_b, W2_w, W2_b, W5_w, W5_b, W6_w,
           W6_b, c1_sw, c1_nw, c1_b, c2_sw, c2_nw, c2_b, W9_w, W9_b,
           W10_w, W10_b, edge_index):
    src = edge_index[0]
    dst = edge_index[1]
    idxp = jnp.stack([src.reshape(_E // _CK, _CK),
                      dst.reshape(_E // _CK, _CK)], axis=1)
    row = lambda v: v.reshape(1, -1)

    n_lo, n_hi = _enc_nodes(C, Fn, W1_w.T, row(W1_b), W2_w.T, row(W2_b))
    e_lo, e_hi = _enc_edges(A, SP1, SP0, W5_w.T, row(W5_b), W6_w.T,
                            row(W6_b))
    dg = _deg_call(dst, jnp.ones((_CK, _H), _f32))
    r1lo, r1hi = _seg_call(n_lo, n_hi, e_lo, e_hi, idxp)
    m2lo, m2hi, s2 = _node1(n_lo, n_hi, r1lo, r1hi, dg,
                            c1_sw.T, c1_nw.T, row(c1_b),
                            c2_nw.T, c2_sw.T, row(c2_b))
    r2lo, r2hi = _seg_call(m2lo, m2hi, e_lo, e_hi, idxp)
    m3lo, m3hi, s3 = _node2(s2, r2lo, r2hi, dg, n_lo, n_hi,
                            c2_nw.T, c2_sw.T, row(c2_b))
    r3lo, r3hi = _seg_call(m3lo, m3hi, e_lo, e_hi, idxp)
    palo, pahi, pblo, pbhi = _node3(s3, r3lo, r3hi, dg,
                                    W9_w[:, :_D].T, W9_w[:, _D:].T,
                                    row(W9_b))
    w10 = W10_w[0]
    p0, p1 = _edge_call(palo, pahi, pblo, pbhi, w10[:_H], w10[_H:], idxp)
    return _final(p0, p1, W10_b.reshape(1, 1))


# speedup vs baseline: 3.3014x; 1.0713x over previous
import functools

import jax
import jax.numpy as jnp
from jax import lax
from jax.experimental import pallas as pl
from jax.experimental.pallas import tpu as pltpu
from jax.experimental.pallas import tpu_sc as plsc

_N = 10000
_E = 160000
_D = 256
_H = 128
_NP = 10240
_RPS = 640
_NSUB = 16
_EPS = _E // _NSUB
_CK = 80
_NCH = _EPS // _CK
_EPW = _E // 32
_CKD = 40
_NCHD = _EPW // _CKD

_f32 = jnp.float32


def _enc_body(a, s1, s0, c, fn, w5c, b5, w6t, b6, w1c, b1, w2t, b2,
              elo, ehi, nlo, nhi):
    he = (a[...] * w5c[0:1, :] + s1[...] * w5c[1:2, :] + s0[...] * w5c[2:3, :]
          + b5[...])
    he = jnp.maximum(he, 0.0)
    oe = jnp.dot(he, w6t[...], preferred_element_type=_f32) + b6[...]
    elo[...] = oe[:, :_H]
    ehi[...] = oe[:, _H:]
    hn = c[...] * w1c[0:1, :] + fn[...] * w1c[1:2, :] + b1[...]
    hn = jnp.maximum(hn, 0.0)
    on = jnp.dot(hn, w2t[...], preferred_element_type=_f32) + b2[...]
    nlo[...] = on[:, :_H]
    nhi[...] = on[:, _H:]


def _node1_body(nlo, nhi, rlo, rhi, dg, swt, nwt, b1, nw2t, sw2t, b2,
                m2lo, m2hi, s2):
    n = jnp.concatenate([nlo[...], nhi[...]], axis=1)
    raw = jnp.concatenate([rlo[...], rhi[...]], axis=1)
    deg = jnp.maximum(dg[:, :1], 1.0)
    neigh = jnp.dot(raw / deg, nwt[...], preferred_element_type=_f32)
    h1 = jnp.maximum(
        jnp.dot(n, swt[...], preferred_element_type=_f32) + neigh + b1[...],
        0.0)
    hcat = jnp.concatenate([h1, n], axis=1)
    m2 = jnp.dot(hcat, nw2t[...], preferred_element_type=_f32)
    m2lo[...] = m2[:, :_H]
    m2hi[...] = m2[:, _H:]
    s2[...] = jnp.dot(hcat, sw2t[...], preferred_element_type=_f32) + b2[...]


def _node2_body(s2, rlo, rhi, dg, nlo, nhi, nw2t, sw2t, b2,
                m3lo, m3hi, s3):
    raw = jnp.concatenate([rlo[...], rhi[...]], axis=1)
    deg = jnp.maximum(dg[:, :1], 1.0)
    h2 = jnp.maximum(s2[...] + raw / deg, 0.0)
    n = jnp.concatenate([nlo[...], nhi[...]], axis=1)
    hcat = jnp.concatenate([h2, n], axis=1)
    m3 = jnp.dot(hcat, nw2t[...], preferred_element_type=_f32)
    m3lo[...] = m3[:, :_H]
    m3hi[...] = m3[:, _H:]
    s3[...] = jnp.dot(hcat, sw2t[...], preferred_element_type=_f32) + b2[...]


def _node3_body(s3, rlo, rhi, dg, w9at, w9bt, b9,
                palo, pahi, pblo, pbhi):
    raw = jnp.concatenate([rlo[...], rhi[...]], axis=1)
    deg = jnp.maximum(dg[:, :1], 1.0)
    h3 = s3[...] + raw / deg
    pa = jnp.dot(h3, w9at[...], preferred_element_type=_f32) + b9[...]
    pb = jnp.dot(h3, w9bt[...], preferred_element_type=_f32)
    palo[...] = pa[:, :_H]
    pahi[...] = pa[:, _H:]
    pblo[...] = pb[:, :_H]
    pbhi[...] = pb[:, _H:]


def _final_body(p0, p1, b10, o):
    o[...] = jnp.sum(p0[...] + p1[...], axis=1, keepdims=True) + b10[0, 0]


def _full(shape):
    return pl.BlockSpec(shape, lambda i: tuple(0 for _ in shape))


def _rows(bs, w):
    return pl.BlockSpec((bs, w), lambda i: (i, 0))


def _deg_body(dst, ones_h, out0, acc, ones_v, ix0, ix1, sem_d0, sem_d1):
    cid = lax.axis_index("c")
    sid = lax.axis_index("s")

    def zrow(i, _):
        for j in range(8):
            ones_v[i, pl.ds(j * 16, 16)] = jnp.zeros((16,), _f32)
        return 0
    lax.fori_loop(0, 128, zrow, 0)

    def zacc(k, _):
        pltpu.sync_copy(ones_v, acc.at[pl.ds(sid * _RPS + k * 128, 128)])
        return 0
    lax.fori_loop(0, _RPS // 128, zacc, 0)
    pltpu.sync_copy(ones_h, ones_v)
    plsc.subcore_barrier()

    nch = 78
    c0 = sid * nch
    bufs = (ix0, ix1)
    sems = (sem_d0, sem_d1)

    def load(c, b):
        pltpu.sync_copy(dst.at[pl.ds(c * 128, 128)], bufs[b])

    def scat(b):
        pltpu.async_copy(ones_v, acc.at[bufs[b]], sems[b], add=True)

    def drain(b):
        pltpu.make_async_copy(ones_v, acc.at[bufs[b]], sems[b]).wait()

    load(c0, 0)
    scat(0)
    load(c0 + 1, 1)
    scat(1)

    def pair(k, _):
        c = c0 + 2 * k
        drain(0)
        load(c, 0)
        scat(0)
        drain(1)
        load(c + 1, 1)
        scat(1)
        return 0
    lax.fori_loop(1, nch // 2, pair, 0)

    @pl.when(sid < 2)
    def _():
        drain(0)
        load(16 * nch + sid, 0)
        scat(0)
    drain(0)
    drain(1)
    plsc.subcore_barrier()

    @pl.when(cid == 0)
    def _():
        pltpu.sync_copy(acc.at[pl.ds(sid * _RPS, _RPS)],
                        out0.at[pl.ds(sid * _RPS, _RPS)])


def _seg_body(flo, fhi, elo, ehi, idxp, out_lo, out_hi,
              acc, rows_v, e_v, ix2, sem_g, sem_e,
              rows_v2, e_v2, ix2b, sem_g2, sem_e2):
    cid = lax.axis_index("c")
    sid = lax.axis_index("s")

    def zrow(i, _):
        for j in range(8):
            rows_v[i, pl.ds(j * 16, 16)] = jnp.zeros((16,), _f32)
        return 0
    lax.fori_loop(0, _CK, zrow, 0)

    def zacc(k, _):
        pltpu.sync_copy(rows_v, acc.at[pl.ds(sid * _RPS + k * _CK, _CK)])
        return 0
    lax.fori_loop(0, _RPS // _CK, zacc, 0)
    plsc.subcore_barrier()

    def run(feat_ref, e_ref):
        bufs = ((rows_v, e_v, ix2, sem_g, sem_e),
                (rows_v2, e_v2, ix2b, sem_g2, sem_e2))

        def start(ch, b):
            rows, ev, ix, sg, se = bufs[b]
            base = sid * _EPS + ch * _CK
            pltpu.sync_copy(idxp.at[sid * _NCH + ch], ix)
            pltpu.async_copy(feat_ref.at[ix.at[0]], rows, sg)
            pltpu.async_copy(e_ref.at[pl.ds(base, _CK)], ev, se)

        def finish(ch, b):
            rows, ev, ix, sg, se = bufs[b]
            base = sid * _EPS + ch * _CK
            pltpu.make_async_copy(feat_ref.at[ix.at[0]], rows, sg).wait()
            pltpu.make_async_copy(e_ref.at[pl.ds(base, _CK)], ev, se).wait()

            @plsc.parallel_loop(0, _CK, unroll=2)
            def _mul(i):
                for j in range(8):
                    s = pl.ds(j * 16, 16)
                    rows[i, s] = rows[i, s] * ev[i, s]
            pltpu.sync_copy(rows, acc.at[ix.at[1]], add=True)

        start(0, 0)

        def pair(k, _):
            ch = 2 * k
            start(ch + 1, 1)
            finish(ch, 0)
            start(ch + 2, 0)
            finish(ch + 1, 1)
            return 0
        lax.fori_loop(0, (_NCH - 1) // 2, pair, 0)
        finish(_NCH - 1, 0)

    @pl.when(cid == 0)
    def _():
        run(flo, elo)

    @pl.when(cid == 1)
    def _():
        run(fhi, ehi)

    plsc.subcore_barrier()

    @pl.when(cid == 0)
    def _():
        pltpu.sync_copy(acc.at[pl.ds(sid * _RPS, _RPS)],
                        out_lo.at[pl.ds(sid * _RPS, _RPS)])

    @pl.when(cid == 1)
    def _():
        pltpu.sync_copy(acc.at[pl.ds(sid * _RPS, _RPS)],
                        out_hi.at[pl.ds(sid * _RPS, _RPS)])


def _edge_body(palo, pahi, pblo, pbhi, w10lo, w10hi, idxp, out0, out1,
               a_v, b_v, w_v, o_v, ix2, sem_a, sem_b,
               a_v2, b_v2, o_v2, ix2b, sem_a2, sem_b2, sem_o, sem_o2):
    cid = lax.axis_index("c")
    sid = lax.axis_index("s")

    def run(pa_ref, pb_ref, w_ref, out_ref):
        pltpu.sync_copy(w_ref, w_v)
        bufs = ((a_v, b_v, o_v, ix2, sem_a, sem_b, sem_o),
                (a_v2, b_v2, o_v2, ix2b, sem_a2, sem_b2, sem_o2))

        def start(ch, b):
            av, bv, ov, ix, sa, sb, so = bufs[b]
            pltpu.sync_copy(idxp.at[sid * _NCH + ch], ix)
            pltpu.async_copy(pa_ref.at[ix.at[0]], av, sa)
            pltpu.async_copy(pb_ref.at[ix.at[1]], bv, sb)

        def finish(ch, b):
            av, bv, ov, ix, sa, sb, so = bufs[b]
            base = sid * _EPS + ch * _CK
            pltpu.make_async_copy(pa_ref.at[ix.at[0]], av, sa).wait()
            pltpu.make_async_copy(pb_ref.at[ix.at[1]], bv, sb).wait()

            @pl.when(ch >= 2)
            def _():
                pltpu.make_async_copy(ov, out_ref.at[pl.ds(0, _CK)], so).wait()

            @plsc.parallel_loop(0, _CK, unroll=2)
            def _edot(i):
                acc = jnp.zeros((16,), _f32)
                for j in range(8):
                    s = pl.ds(j * 16, 16)
                    t = jnp.maximum(av[i, s] + bv[i, s], 0.0)
                    acc = acc + t * w_v[s]
                ov[i, :] = acc
            pltpu.async_copy(ov, out_ref.at[pl.ds(base, _CK)], so)

        start(0, 0)

        def pair(k, _):
            ch = 2 * k
            start(ch + 1, 1)
            finish(ch, 0)
            start(ch + 2, 0)
            finish(ch + 1, 1)
            return 0
        lax.fori_loop(0, (_NCH - 1) // 2, pair, 0)
        finish(_NCH - 1, 0)
        pltpu.make_async_copy(o_v, out_ref.at[pl.ds(0, _CK)], sem_o).wait()
        pltpu.make_async_copy(o_v2, out_ref.at[pl.ds(0, _CK)], sem_o2).wait()

    @pl.when(cid == 0)
    def _():
        run(palo, pblo, w10lo, out0)

    @pl.when(cid == 1)
    def _():
        run(pahi, pbhi, w10hi, out1)


def _sc_mesh():
    return plsc.VectorSubcoreMesh(core_axis_name="c", subcore_axis_name="s")


def _deg_call(dst, ones_h):
    return pl.kernel(
        _deg_body,
        out_type=jax.ShapeDtypeStruct((_NP, _H), _f32),
        mesh=_sc_mesh(),
        scratch_types=[pltpu.VMEM_SHARED((_NP, _H), _f32),
                       pltpu.VMEM((128, _H), _f32),
                       pltpu.VMEM((128,), jnp.int32),
                       pltpu.VMEM((128,), jnp.int32),
                       pltpu.SemaphoreType.DMA,
                       pltpu.SemaphoreType.DMA],
    )(dst, ones_h)


def _seg_call(flo, fhi, elo, ehi, idxp):
    return pl.kernel(
        _seg_body,
        out_type=[jax.ShapeDtypeStruct((_NP, _H), _f32),
                  jax.ShapeDtypeStruct((_NP, _H), _f32)],
        mesh=_sc_mesh(),
        scratch_types=[pltpu.VMEM_SHARED((_NP, _H), _f32),
                       pltpu.VMEM((_CK, _H), _f32),
                       pltpu.VMEM((_CK, _H), _f32),
                       pltpu.VMEM((2, _CK), jnp.int32),
                       pltpu.SemaphoreType.DMA,
                       pltpu.SemaphoreType.DMA,
                       pltpu.VMEM((_CK, _H), _f32),
                       pltpu.VMEM((_CK, _H), _f32),
                       pltpu.VMEM((2, _CK), jnp.int32),
                       pltpu.SemaphoreType.DMA,
                       pltpu.SemaphoreType.DMA],
    )(flo, fhi, elo, ehi, idxp)


def _edge_call(palo, pahi, pblo, pbhi, w10lo, w10hi, idxp):
    return pl.kernel(
        _edge_body,
        out_type=[jax.ShapeDtypeStruct((_E, 16), _f32),
                  jax.ShapeDtypeStruct((_E, 16), _f32)],
        mesh=_sc_mesh(),
        scratch_types=[pltpu.VMEM((_CK, _H), _f32),
                       pltpu.VMEM((_CK, _H), _f32),
                       pltpu.VMEM((_H,), _f32),
                       pltpu.VMEM((_CK, 16), _f32),
                       pltpu.VMEM((2, _CK), jnp.int32),
                       pltpu.SemaphoreType.DMA,
                       pltpu.SemaphoreType.DMA,
                       pltpu.VMEM((_CK, _H), _f32),
                       pltpu.VMEM((_CK, _H), _f32),
                       pltpu.VMEM((_CK, 16), _f32),
                       pltpu.VMEM((2, _CK), jnp.int32),
                       pltpu.SemaphoreType.DMA,
                       pltpu.SemaphoreType.DMA,
                       pltpu.SemaphoreType.DMA,
                       pltpu.SemaphoreType.DMA],
    )(palo, pahi, pblo, pbhi, w10lo, w10hi, idxp)

_BN = 2000
_BE = 3200


_BNE = _N // (_E // _BE)


def _enc_all(a, s1, s0, c, fn, w5c, b5, w6t, b6, w1c, b1, w2t, b2):
    return pl.pallas_call(
        _enc_body,
        grid=(_E // _BE,),
        in_specs=[_rows(_BE, 1), _rows(_BE, 1), _rows(_BE, 1),
                  _rows(_BNE, 1), _rows(_BNE, 1),
                  _full((3, _D)), _full((1, _D)), _full((_D, _D)),
                  _full((1, _D)),
                  _full((2, _D)), _full((1, _D)), _full((_D, _D)),
                  _full((1, _D))],
        out_specs=[_rows(_BE, _H), _rows(_BE, _H),
                   _rows(_BNE, _H), _rows(_BNE, _H)],
        out_shape=[jax.ShapeDtypeStruct((_E, _H), _f32),
                   jax.ShapeDtypeStruct((_E, _H), _f32),
                   jax.ShapeDtypeStruct((_N, _H), _f32),
                   jax.ShapeDtypeStruct((_N, _H), _f32)],
    )(a, s1, s0, c, fn, w5c, b5, w6t, b6, w1c, b1, w2t, b2)


def _node1(nlo, nhi, rlo, rhi, dg, swt, nwt, b1, nw2t, sw2t, b2):
    return pl.pallas_call(
        _node1_body,
        grid=(_N // _BN,),
        in_specs=[_rows(_BN, _H), _rows(_BN, _H), _rows(_BN, _H),
                  _rows(_BN, _H), _rows(_BN, _H),
                  _full((_D, _D)), _full((_D, _D)), _full((1, _D)),
                  _full((2 * _D, _D)), _full((2 * _D, _D)), _full((1, _D))],
        out_specs=[_rows(_BN, _H), _rows(_BN, _H), _rows(_BN, _D)],
        out_shape=[jax.ShapeDtypeStruct((_N, _H), _f32),
                   jax.ShapeDtypeStruct((_N, _H), _f32),
                   jax.ShapeDtypeStruct((_N, _D), _f32)],
    )(nlo, nhi, rlo, rhi, dg, swt, nwt, b1, nw2t, sw2t, b2)


def _node2(s2, rlo, rhi, dg, nlo, nhi, nw2t, sw2t, b2):
    return pl.pallas_call(
        _node2_body,
        grid=(_N // _BN,),
        in_specs=[_rows(_BN, _D), _rows(_BN, _H), _rows(_BN, _H),
                  _rows(_BN, _H), _rows(_BN, _H),
                  _rows(_BN, _H), _full((2 * _D, _D)), _full((2 * _D, _D)),
                  _full((1, _D))],
        out_specs=[_rows(_BN, _H), _rows(_BN, _H), _rows(_BN, _D)],
        out_shape=[jax.ShapeDtypeStruct((_N, _H), _f32),
                   jax.ShapeDtypeStruct((_N, _H), _f32),
                   jax.ShapeDtypeStruct((_N, _D), _f32)],
    )(s2, rlo, rhi, dg, nlo, nhi, nw2t, sw2t, b2)


def _node3(s3, rlo, rhi, dg, w9at, w9bt, b9):
    return pl.pallas_call(
        _node3_body,
        grid=(_N // _BN,),
        in_specs=[_rows(_BN, _D), _rows(_BN, _H), _rows(_BN, _H),
                  _rows(_BN, _H),
                  _full((_D, _D)), _full((_D, _D)), _full((1, _D))],
        out_specs=[_rows(_BN, _H), _rows(_BN, _H),
                   _rows(_BN, _H), _rows(_BN, _H)],
        out_shape=[jax.ShapeDtypeStruct((_N, _H), _f32),
                   jax.ShapeDtypeStruct((_N, _H), _f32),
                   jax.ShapeDtypeStruct((_N, _H), _f32),
                   jax.ShapeDtypeStruct((_N, _H), _f32)],
    )(s3, rlo, rhi, dg, w9at, w9bt, b9)


def _final(p0, p1, b10):
    return pl.pallas_call(
        _final_body,
        grid=(_E // _BE,),
        in_specs=[_rows(_BE, 16), _rows(_BE, 16), _full((1, 1))],
        out_specs=[_rows(_BE, 1)],
        out_shape=[jax.ShapeDtypeStruct((_E, 1), _f32)],
    )(p0, p1, b10)[0].reshape(_E)


def kernel(C, Fn, A, SP1, SP0, W1_w, W1_b, W2_w, W2_b, W5_w, W5_b, W6_w,
           W6_b, c1_sw, c1_nw, c1_b, c2_sw, c2_nw, c2_b, W9_w, W9_b,
           W10_w, W10_b, edge_index):
    src = edge_index[0]
    dst = edge_index[1]
    idxp = jnp.stack([src.reshape(_E // _CK, _CK),
                      dst.reshape(_E // _CK, _CK)], axis=1)
    row = lambda v: v.reshape(1, -1)

    e_lo, e_hi, n_lo, n_hi = _enc_all(
        A, SP1, SP0, C, Fn, W5_w.T, row(W5_b), W6_w.T, row(W6_b),
        W1_w.T, row(W1_b), W2_w.T, row(W2_b))
    dg = _deg_call(dst, jnp.ones((128, _H), _f32))
    r1lo, r1hi = _seg_call(n_lo, n_hi, e_lo, e_hi, idxp)
    m2lo, m2hi, s2 = _node1(n_lo, n_hi, r1lo, r1hi, dg,
                            c1_sw.T, c1_nw.T, row(c1_b),
                            c2_nw.T, c2_sw.T, row(c2_b))
    r2lo, r2hi = _seg_call(m2lo, m2hi, e_lo, e_hi, idxp)
    m3lo, m3hi, s3 = _node2(s2, r2lo, r2hi, dg, n_lo, n_hi,
                            c2_nw.T, c2_sw.T, row(c2_b))
    r3lo, r3hi = _seg_call(m3lo, m3hi, e_lo, e_hi, idxp)
    palo, pahi, pblo, pbhi = _node3(s3, r3lo, r3hi, dg,
                                    W9_w[:, :_D].T, W9_w[:, _D:].T,
                                    row(W9_b))
    w10 = W10_w[0]
    p0, p1 = _edge_call(palo, pahi, pblo, pbhi, w10[:_H], w10[_H:], idxp)
    return _final(p0, p1, W10_b.reshape(1, 1))


# speedup vs baseline: 3.3666x; 1.0198x over previous
import functools

import jax
import jax.numpy as jnp
from jax import lax
from jax.experimental import pallas as pl
from jax.experimental.pallas import tpu as pltpu
from jax.experimental.pallas import tpu_sc as plsc

_N = 10000
_E = 160000
_D = 256
_H = 128
_NP = 10240
_RPS = 640
_NSUB = 16
_EPS = _E // _NSUB
_CK = 80
_NCH = _EPS // _CK
_EPW = _E // 32
_CKD = 40
_NCHD = _EPW // _CKD

_f32 = jnp.float32


def _enc_body(a, s1, s0, c, fn, w5c, b5, w6t, b6, w1c, b1, w2t, b2,
              elo, ehi, nlo, nhi):
    he = (a[...] * w5c[0:1, :] + s1[...] * w5c[1:2, :] + s0[...] * w5c[2:3, :]
          + b5[...])
    he = jnp.maximum(he, 0.0)
    oe = jnp.dot(he, w6t[...], preferred_element_type=_f32) + b6[...]
    elo[...] = oe[:, :_H]
    ehi[...] = oe[:, _H:]
    hn = c[...] * w1c[0:1, :] + fn[...] * w1c[1:2, :] + b1[...]
    hn = jnp.maximum(hn, 0.0)
    on = jnp.dot(hn, w2t[...], preferred_element_type=_f32) + b2[...]
    nlo[...] = on[:, :_H]
    nhi[...] = on[:, _H:]


def _node1_body(nlo, nhi, rlo, rhi, dg, swt, nwt, b1, nw2t, sw2t, b2,
                m2lo, m2hi, s2):
    n = jnp.concatenate([nlo[...], nhi[...]], axis=1)
    raw = jnp.concatenate([rlo[...], rhi[...]], axis=1)
    deg = jnp.maximum(dg[:, :1], 1.0)
    neigh = jnp.dot(raw / deg, nwt[...], preferred_element_type=_f32)
    h1 = jnp.maximum(
        jnp.dot(n, swt[...], preferred_element_type=_f32) + neigh + b1[...],
        0.0)
    hcat = jnp.concatenate([h1, n], axis=1)
    m2 = jnp.dot(hcat, nw2t[...], preferred_element_type=_f32)
    m2lo[...] = m2[:, :_H]
    m2hi[...] = m2[:, _H:]
    s2[...] = jnp.dot(hcat, sw2t[...], preferred_element_type=_f32) + b2[...]


def _node2_body(s2, rlo, rhi, dg, nlo, nhi, nw2t, sw2t, b2,
                m3lo, m3hi, s3):
    raw = jnp.concatenate([rlo[...], rhi[...]], axis=1)
    deg = jnp.maximum(dg[:, :1], 1.0)
    h2 = jnp.maximum(s2[...] + raw / deg, 0.0)
    n = jnp.concatenate([nlo[...], nhi[...]], axis=1)
    hcat = jnp.concatenate([h2, n], axis=1)
    m3 = jnp.dot(hcat, nw2t[...], preferred_element_type=_f32)
    m3lo[...] = m3[:, :_H]
    m3hi[...] = m3[:, _H:]
    s3[...] = jnp.dot(hcat, sw2t[...], preferred_element_type=_f32) + b2[...]


def _node3_body(s3, rlo, rhi, dg, w9at, w9bt, b9,
                palo, pahi, pblo, pbhi):
    raw = jnp.concatenate([rlo[...], rhi[...]], axis=1)
    deg = jnp.maximum(dg[:, :1], 1.0)
    h3 = s3[...] + raw / deg
    pa = jnp.dot(h3, w9at[...], preferred_element_type=_f32) + b9[...]
    pb = jnp.dot(h3, w9bt[...], preferred_element_type=_f32)
    palo[...] = pa[:, :_H]
    pahi[...] = pa[:, _H:]
    pblo[...] = pb[:, :_H]
    pbhi[...] = pb[:, _H:]


def _final_body(p0, p1, b10, o):
    o[...] = jnp.sum(p0[...] + p1[...], axis=1, keepdims=True) + b10[0, 0]


def _full(shape):
    return pl.BlockSpec(shape, lambda i: tuple(0 for _ in shape))


def _rows(bs, w):
    return pl.BlockSpec((bs, w), lambda i: (i, 0))


def _deg_body(dst, ones_h, out0, acc, ones_v, ix0, ix1, sem_d0, sem_d1):
    cid = lax.axis_index("c")
    sid = lax.axis_index("s")

    def zrow(i, _):
        for j in range(8):
            ones_v[i, pl.ds(j * 16, 16)] = jnp.zeros((16,), _f32)
        return 0
    lax.fori_loop(0, 128, zrow, 0)

    def zacc(k, _):
        pltpu.sync_copy(ones_v, acc.at[pl.ds(sid * _RPS + k * 128, 128)])
        return 0
    lax.fori_loop(0, _RPS // 128, zacc, 0)
    pltpu.sync_copy(ones_h, ones_v)
    plsc.subcore_barrier()

    nch = 78
    c0 = sid * nch
    bufs = (ix0, ix1)
    sems = (sem_d0, sem_d1)

    def load(c, b):
        pltpu.sync_copy(dst.at[pl.ds(c * 128, 128)], bufs[b])

    def scat(b):
        pltpu.async_copy(ones_v, acc.at[bufs[b]], sems[b], add=True)

    def drain(b):
        pltpu.make_async_copy(ones_v, acc.at[bufs[b]], sems[b]).wait()

    load(c0, 0)
    scat(0)
    load(c0 + 1, 1)
    scat(1)

    def pair(k, _):
        c = c0 + 2 * k
        drain(0)
        load(c, 0)
        scat(0)
        drain(1)
        load(c + 1, 1)
        scat(1)
        return 0
    lax.fori_loop(1, nch // 2, pair, 0)

    @pl.when(sid < 2)
    def _():
        drain(0)
        load(16 * nch + sid, 0)
        scat(0)
    drain(0)
    drain(1)
    plsc.subcore_barrier()

    @pl.when(cid == 0)
    def _():
        pltpu.sync_copy(acc.at[pl.ds(sid * _RPS, _RPS)],
                        out0.at[pl.ds(sid * _RPS, _RPS)])


def _seg_body(flo, fhi, elo, ehi, idxp, out_lo, out_hi,
              acc, rows_v, e_v, ix2, sem_g, sem_e, sem_i,
              rows_v2, e_v2, ix2b, sem_g2, sem_e2, sem_i2):
    cid = lax.axis_index("c")
    sid = lax.axis_index("s")

    def zrow(i, _):
        for j in range(8):
            rows_v[i, pl.ds(j * 16, 16)] = jnp.zeros((16,), _f32)
        return 0
    lax.fori_loop(0, _CK, zrow, 0)

    def zacc(k, _):
        pltpu.sync_copy(rows_v, acc.at[pl.ds(sid * _RPS + k * _CK, _CK)])
        return 0
    lax.fori_loop(0, _RPS // _CK, zacc, 0)
    plsc.subcore_barrier()

    def run(feat_ref, e_ref):
        bufs = ((rows_v, e_v, ix2, sem_g, sem_e, sem_i),
                (rows_v2, e_v2, ix2b, sem_g2, sem_e2, sem_i2))

        def start(ch, b, first=False):
            rows, ev, ix, sg, se, si = bufs[b]
            base = sid * _EPS + ch * _CK
            if first:
                pltpu.sync_copy(idxp.at[sid * _NCH + ch], ix)
            else:
                pltpu.make_async_copy(idxp.at[0], ix, si).wait()
            pltpu.async_copy(feat_ref.at[ix.at[0]], rows, sg)
            pltpu.async_copy(e_ref.at[pl.ds(base, _CK)], ev, se)

        def finish(ch, b):
            rows, ev, ix, sg, se, si = bufs[b]
            base = sid * _EPS + ch * _CK
            pltpu.make_async_copy(feat_ref.at[ix.at[0]], rows, sg).wait()
            pltpu.make_async_copy(e_ref.at[pl.ds(base, _CK)], ev, se).wait()

            @plsc.parallel_loop(0, _CK, unroll=4)
            def _mul(i):
                for j in range(8):
                    s = pl.ds(j * 16, 16)
                    rows[i, s] = rows[i, s] * ev[i, s]
            pltpu.sync_copy(rows, acc.at[ix.at[1]], add=True)

            @pl.when(ch + 2 < _NCH)
            def _():
                pltpu.async_copy(idxp.at[sid * _NCH + ch + 2], ix, si)

        start(0, 0, first=True)
        start(1, 1, first=True)
        finish(0, 0)
        start(2, 0)
        finish(1, 1)

        def pair(k, _):
            ch = 2 * k
            start(ch + 1, 1)
            finish(ch, 0)
            start(ch + 2, 0)
            finish(ch + 1, 1)
            return 0
        lax.fori_loop(1, (_NCH - 1) // 2, pair, 0)
        finish(_NCH - 1, 0)

    @pl.when(cid == 0)
    def _():
        run(flo, elo)

    @pl.when(cid == 1)
    def _():
        run(fhi, ehi)

    plsc.subcore_barrier()

    @pl.when(cid == 0)
    def _():
        pltpu.sync_copy(acc.at[pl.ds(sid * _RPS, _RPS)],
                        out_lo.at[pl.ds(sid * _RPS, _RPS)])

    @pl.when(cid == 1)
    def _():
        pltpu.sync_copy(acc.at[pl.ds(sid * _RPS, _RPS)],
                        out_hi.at[pl.ds(sid * _RPS, _RPS)])


def _edge_body(palo, pahi, pblo, pbhi, w10lo, w10hi, idxp, out0, out1,
               a_v, b_v, w_v, o_v, ix2, sem_a, sem_b, sem_i,
               a_v2, b_v2, o_v2, ix2b, sem_a2, sem_b2, sem_i2,
               sem_o, sem_o2):
    cid = lax.axis_index("c")
    sid = lax.axis_index("s")

    def run(pa_ref, pb_ref, w_ref, out_ref):
        pltpu.sync_copy(w_ref, w_v)
        bufs = ((a_v, b_v, o_v, ix2, sem_a, sem_b, sem_o, sem_i),
                (a_v2, b_v2, o_v2, ix2b, sem_a2, sem_b2, sem_o2, sem_i2))

        def start(ch, b, first=False):
            av, bv, ov, ix, sa, sb, so, si = bufs[b]
            if first:
                pltpu.sync_copy(idxp.at[sid * _NCH + ch], ix)
            else:
                pltpu.make_async_copy(idxp.at[0], ix, si).wait()
            pltpu.async_copy(pa_ref.at[ix.at[0]], av, sa)
            pltpu.async_copy(pb_ref.at[ix.at[1]], bv, sb)

        def finish(ch, b):
            av, bv, ov, ix, sa, sb, so, si = bufs[b]
            base = sid * _EPS + ch * _CK
            pltpu.make_async_copy(pa_ref.at[ix.at[0]], av, sa).wait()
            pltpu.make_async_copy(pb_ref.at[ix.at[1]], bv, sb).wait()

            @pl.when(ch + 2 < _NCH)
            def _():
                pltpu.async_copy(idxp.at[sid * _NCH + ch + 2], ix, si)

            @pl.when(ch >= 2)
            def _():
                pltpu.make_async_copy(ov, out_ref.at[pl.ds(0, _CK)], so).wait()

            @plsc.parallel_loop(0, _CK, unroll=2)
            def _edot(i):
                acc = jnp.zeros((16,), _f32)
                for j in range(8):
                    s = pl.ds(j * 16, 16)
                    t = jnp.maximum(av[i, s] + bv[i, s], 0.0)
                    acc = acc + t * w_v[s]
                ov[i, :] = acc
            pltpu.async_copy(ov, out_ref.at[pl.ds(base, _CK)], so)

        start(0, 0, first=True)
        start(1, 1, first=True)
        finish(0, 0)
        start(2, 0)
        finish(1, 1)

        def pair(k, _):
            ch = 2 * k
            start(ch + 1, 1)
            finish(ch, 0)
            start(ch + 2, 0)
            finish(ch + 1, 1)
            return 0
        lax.fori_loop(1, (_NCH - 1) // 2, pair, 0)
        finish(_NCH - 1, 0)
        pltpu.make_async_copy(o_v, out_ref.at[pl.ds(0, _CK)], sem_o).wait()
        pltpu.make_async_copy(o_v2, out_ref.at[pl.ds(0, _CK)], sem_o2).wait()

    @pl.when(cid == 0)
    def _():
        run(palo, pblo, w10lo, out0)

    @pl.when(cid == 1)
    def _():
        run(pahi, pbhi, w10hi, out1)


def _sc_mesh():
    return plsc.VectorSubcoreMesh(core_axis_name="c", subcore_axis_name="s")


def _deg_call(dst, ones_h):
    return pl.kernel(
        _deg_body,
        out_type=jax.ShapeDtypeStruct((_NP, _H), _f32),
        mesh=_sc_mesh(),
        scratch_types=[pltpu.VMEM_SHARED((_NP, _H), _f32),
                       pltpu.VMEM((128, _H), _f32),
                       pltpu.VMEM((128,), jnp.int32),
                       pltpu.VMEM((128,), jnp.int32),
                       pltpu.SemaphoreType.DMA,
                       pltpu.SemaphoreType.DMA],
    )(dst, ones_h)


def _seg_call(flo, fhi, elo, ehi, idxp):
    return pl.kernel(
        _seg_body,
        out_type=[jax.ShapeDtypeStruct((_NP, _H), _f32),
                  jax.ShapeDtypeStruct((_NP, _H), _f32)],
        mesh=_sc_mesh(),
        scratch_types=[pltpu.VMEM_SHARED((_NP, _H), _f32),
                       pltpu.VMEM((_CK, _H), _f32),
                       pltpu.VMEM((_CK, _H), _f32),
                       pltpu.VMEM((2, _CK), jnp.int32),
                       pltpu.SemaphoreType.DMA,
                       pltpu.SemaphoreType.DMA,
                       pltpu.SemaphoreType.DMA,
                       pltpu.VMEM((_CK, _H), _f32),
                       pltpu.VMEM((_CK, _H), _f32),
                       pltpu.VMEM((2, _CK), jnp.int32),
                       pltpu.SemaphoreType.DMA,
                       pltpu.SemaphoreType.DMA,
                       pltpu.SemaphoreType.DMA],
    )(flo, fhi, elo, ehi, idxp)


def _edge_call(palo, pahi, pblo, pbhi, w10lo, w10hi, idxp):
    return pl.kernel(
        _edge_body,
        out_type=[jax.ShapeDtypeStruct((_E, 16), _f32),
                  jax.ShapeDtypeStruct((_E, 16), _f32)],
        mesh=_sc_mesh(),
        scratch_types=[pltpu.VMEM((_CK, _H), _f32),
                       pltpu.VMEM((_CK, _H), _f32),
                       pltpu.VMEM((_H,), _f32),
                       pltpu.VMEM((_CK, 16), _f32),
                       pltpu.VMEM((2, _CK), jnp.int32),
                       pltpu.SemaphoreType.DMA,
                       pltpu.SemaphoreType.DMA,
                       pltpu.SemaphoreType.DMA,
                       pltpu.VMEM((_CK, _H), _f32),
                       pltpu.VMEM((_CK, _H), _f32),
                       pltpu.VMEM((_CK, 16), _f32),
                       pltpu.VMEM((2, _CK), jnp.int32),
                       pltpu.SemaphoreType.DMA,
                       pltpu.SemaphoreType.DMA,
                       pltpu.SemaphoreType.DMA,
                       pltpu.SemaphoreType.DMA,
                       pltpu.SemaphoreType.DMA],
    )(palo, pahi, pblo, pbhi, w10lo, w10hi, idxp)

_BN = 2000
_BE = 3200


_BNE = _N // (_E // _BE)


def _enc_all(a, s1, s0, c, fn, w5c, b5, w6t, b6, w1c, b1, w2t, b2):
    return pl.pallas_call(
        _enc_body,
        grid=(_E // _BE,),
        in_specs=[_rows(_BE, 1), _rows(_BE, 1), _rows(_BE, 1),
                  _rows(_BNE, 1), _rows(_BNE, 1),
                  _full((3, _D)), _full((1, _D)), _full((_D, _D)),
                  _full((1, _D)),
                  _full((2, _D)), _full((1, _D)), _full((_D, _D)),
                  _full((1, _D))],
        out_specs=[_rows(_BE, _H), _rows(_BE, _H),
                   _rows(_BNE, _H), _rows(_BNE, _H)],
        out_shape=[jax.ShapeDtypeStruct((_E, _H), _f32),
                   jax.ShapeDtypeStruct((_E, _H), _f32),
                   jax.ShapeDtypeStruct((_N, _H), _f32),
                   jax.ShapeDtypeStruct((_N, _H), _f32)],
    )(a, s1, s0, c, fn, w5c, b5, w6t, b6, w1c, b1, w2t, b2)


def _node1(nlo, nhi, rlo, rhi, dg, swt, nwt, b1, nw2t, sw2t, b2):
    return pl.pallas_call(
        _node1_body,
        grid=(_N // _BN,),
        in_specs=[_rows(_BN, _H), _rows(_BN, _H), _rows(_BN, _H),
                  _rows(_BN, _H), _rows(_BN, _H),
                  _full((_D, _D)), _full((_D, _D)), _full((1, _D)),
                  _full((2 * _D, _D)), _full((2 * _D, _D)), _full((1, _D))],
        out_specs=[_rows(_BN, _H), _rows(_BN, _H), _rows(_BN, _D)],
        out_shape=[jax.ShapeDtypeStruct((_N, _H), _f32),
                   jax.ShapeDtypeStruct((_N, _H), _f32),
                   jax.ShapeDtypeStruct((_N, _D), _f32)],
    )(nlo, nhi, rlo, rhi, dg, swt, nwt, b1, nw2t, sw2t, b2)


def _node2(s2, rlo, rhi, dg, nlo, nhi, nw2t, sw2t, b2):
    return pl.pallas_call(
        _node2_body,
        grid=(_N // _BN,),
        in_specs=[_rows(_BN, _D), _rows(_BN, _H), _rows(_BN, _H),
                  _rows(_BN, _H), _rows(_BN, _H),
                  _rows(_BN, _H), _full((2 * _D, _D)), _full((2 * _D, _D)),
                  _full((1, _D))],
        out_specs=[_rows(_BN, _H), _rows(_BN, _H), _rows(_BN, _D)],
        out_shape=[jax.ShapeDtypeStruct((_N, _H), _f32),
                   jax.ShapeDtypeStruct((_N, _H), _f32),
                   jax.ShapeDtypeStruct((_N, _D), _f32)],
    )(s2, rlo, rhi, dg, nlo, nhi, nw2t, sw2t, b2)


def _node3(s3, rlo, rhi, dg, w9at, w9bt, b9):
    return pl.pallas_call(
        _node3_body,
        grid=(_N // _BN,),
        in_specs=[_rows(_BN, _D), _rows(_BN, _H), _rows(_BN, _H),
                  _rows(_BN, _H),
                  _full((_D, _D)), _full((_D, _D)), _full((1, _D))],
        out_specs=[_rows(_BN, _H), _rows(_BN, _H),
                   _rows(_BN, _H), _rows(_BN, _H)],
        out_shape=[jax.ShapeDtypeStruct((_N, _H), _f32),
                   jax.ShapeDtypeStruct((_N, _H), _f32),
                   jax.ShapeDtypeStruct((_N, _H), _f32),
                   jax.ShapeDtypeStruct((_N, _H), _f32)],
    )(s3, rlo, rhi, dg, w9at, w9bt, b9)


def _final(p0, p1, b10):
    return pl.pallas_call(
        _final_body,
        grid=(_E // _BE,),
        in_specs=[_rows(_BE, 16), _rows(_BE, 16), _full((1, 1))],
        out_specs=[_rows(_BE, 1)],
        out_shape=[jax.ShapeDtypeStruct((_E, 1), _f32)],
    )(p0, p1, b10)[0].reshape(_E)


def kernel(C, Fn, A, SP1, SP0, W1_w, W1_b, W2_w, W2_b, W5_w, W5_b, W6_w,
           W6_b, c1_sw, c1_nw, c1_b, c2_sw, c2_nw, c2_b, W9_w, W9_b,
           W10_w, W10_b, edge_index):
    src = edge_index[0]
    dst = edge_index[1]
    idxp = jnp.stack([src.reshape(_E // _CK, _CK),
                      dst.reshape(_E // _CK, _CK)], axis=1)
    row = lambda v: v.reshape(1, -1)

    e_lo, e_hi, n_lo, n_hi = _enc_all(
        A, SP1, SP0, C, Fn, W5_w.T, row(W5_b), W6_w.T, row(W6_b),
        W1_w.T, row(W1_b), W2_w.T, row(W2_b))
    dg = _deg_call(dst, jnp.ones((128, _H), _f32))
    r1lo, r1hi = _seg_call(n_lo, n_hi, e_lo, e_hi, idxp)
    m2lo, m2hi, s2 = _node1(n_lo, n_hi, r1lo, r1hi, dg,
                            c1_sw.T, c1_nw.T, row(c1_b),
                            c2_nw.T, c2_sw.T, row(c2_b))
    r2lo, r2hi = _seg_call(m2lo, m2hi, e_lo, e_hi, idxp)
    m3lo, m3hi, s3 = _node2(s2, r2lo, r2hi, dg, n_lo, n_hi,
                            c2_nw.T, c2_sw.T, row(c2_b))
    r3lo, r3hi = _seg_call(m3lo, m3hi, e_lo, e_hi, idxp)
    palo, pahi, pblo, pbhi = _node3(s3, r3lo, r3hi, dg,
                                    W9_w[:, :_D].T, W9_w[:, _D:].T,
                                    row(W9_b))
    w10 = W10_w[0]
    p0, p1 = _edge_call(palo, pahi, pblo, pbhi, w10[:_H], w10[_H:], idxp)
    return _final(p0, p1, W10_b.reshape(1, 1))
